# baseline (speedup 1.0000x reference)
import functools

import jax
import jax.numpy as jnp
from jax import lax
from jax.experimental import pallas as pl
from jax.experimental.pallas import tpu as pltpu
from jax.experimental.pallas import tpu_sc as plsc

_VOCAB = 100000
_D = 128
_ROWS = 4096
_SEQ = 50

_info = plsc.get_sparse_core_info()
_NC = _info.num_cores
_NS = _info.num_subcores
_NW = _NC * _NS
_RPW = _ROWS // _NW
_N_CHUNKS = _SEQ
_NB = 5
_GROUPS = _N_CHUNKS // _NB
_REM = _N_CHUNKS - _GROUPS * _NB

_mesh = plsc.VectorSubcoreMesh(core_axis_name="c", subcore_axis_name="s")


@functools.partial(
    pl.kernel,
    mesh=_mesh,
    out_type=jax.ShapeDtypeStruct((_SEQ, _ROWS, _D), jnp.float32),
    scratch_types=[
        pltpu.VMEM((_N_CHUNKS, _RPW), jnp.int32),
        pltpu.VMEM((_NB, _RPW, _D), jnp.float32),
    ]
    + [pltpu.SemaphoreType.DMA] * (2 * _NB),
)
def _sc_gather(idx_hbm, table_hbm, out_hbm, idx_v, rows_v, *sems):
    gsems = sems[:_NB]
    ssems = sems[_NB:]
    wid = lax.axis_index("s") * _NC + lax.axis_index("c")
    i_base = wid * _RPW
    pltpu.sync_copy(idx_hbm.at[wid], idx_v)

    for b in range(_NB):
        pltpu.async_copy(table_hbm.at[idx_v.at[b]], rows_v.at[b], gsems[b])

    def group(g, carry):
        for b in range(_NB):
            j = g * _NB + b
            pltpu.make_async_copy(
                table_hbm.at[idx_v.at[j]], rows_v.at[b], gsems[b]
            ).wait()
            pltpu.async_copy(
                rows_v.at[b],
                out_hbm.at[j, pl.ds(i_base, _RPW)],
                ssems[b],
            )

        @pl.when(g < _GROUPS - 1)
        def _():
            for b in range(_NB):
                jn = (g + 1) * _NB + b
                pltpu.make_async_copy(
                    rows_v.at[b], out_hbm.at[0, pl.ds(i_base, _RPW)], ssems[b]
                ).wait()
                pltpu.async_copy(
                    table_hbm.at[idx_v.at[jn]], rows_v.at[b], gsems[b]
                )

        return carry

    lax.fori_loop(0, _GROUPS, group, 0)

    for b in range(_REM):
        jn = _GROUPS * _NB + b
        pltpu.make_async_copy(
            rows_v.at[b], out_hbm.at[0, pl.ds(i_base, _RPW)], ssems[b]
        ).wait()
        pltpu.async_copy(table_hbm.at[idx_v.at[jn]], rows_v.at[b], gsems[b])
    for b in range(_REM):
        jn = _GROUPS * _NB + b
        pltpu.make_async_copy(
            table_hbm.at[idx_v.at[jn]], rows_v.at[b], gsems[b]
        ).wait()
        pltpu.async_copy(
            rows_v.at[b], out_hbm.at[jn, pl.ds(i_base, _RPW)], ssems[b]
        )

    for b in range(_NB):
        pltpu.make_async_copy(
            rows_v.at[b], out_hbm.at[0, pl.ds(i_base, _RPW)], ssems[b]
        ).wait()


def kernel(x, embedding_weights):
    idx = x.astype(jnp.int32).reshape(_NW, _RPW, _SEQ).transpose(0, 2, 1)
    out = _sc_gather(idx, embedding_weights)
    return out.transpose(1, 0, 2)


# speedup vs baseline: 1.0223x; 1.0223x over previous
import functools

import jax
import jax.numpy as jnp
from jax import lax
from jax.experimental import pallas as pl
from jax.experimental.pallas import tpu as pltpu
from jax.experimental.pallas import tpu_sc as plsc

_VOCAB = 100000
_D = 128
_ROWS = 4096
_SEQ = 50

_info = plsc.get_sparse_core_info()
_NC = _info.num_cores
_NS = _info.num_subcores
_NW = _NC * _NS
_RPW = _ROWS // _NW
_CPP = 2
_CH = _RPW // _CPP
_N_CHUNKS = _SEQ * _CPP
_NB = 10
_GROUPS = _N_CHUNKS // _NB
_REM = _N_CHUNKS - _GROUPS * _NB

_mesh = plsc.VectorSubcoreMesh(core_axis_name="c", subcore_axis_name="s")


@functools.partial(
    pl.kernel,
    mesh=_mesh,
    out_type=jax.ShapeDtypeStruct((_SEQ, _ROWS, _D), jnp.float32),
    scratch_types=[
        pltpu.VMEM((_N_CHUNKS, _CH), jnp.int32),
        pltpu.VMEM((_NB, _CH, _D), jnp.float32),
    ]
    + [pltpu.SemaphoreType.DMA] * (2 * _NB),
)
def _sc_gather(idx_hbm, table_hbm, out_hbm, idx_v, rows_v, *sems):
    gsems = sems[:_NB]
    ssems = sems[_NB:]
    wid = lax.axis_index("s") * _NC + lax.axis_index("c")
    i_base = wid * _RPW
    pltpu.sync_copy(idx_hbm.at[wid], idx_v)

    for b in range(_NB):
        pltpu.async_copy(table_hbm.at[idx_v.at[b]], rows_v.at[b], gsems[b])

    def group(g, carry):
        for b in range(_NB):
            j = g * _NB + b
            pltpu.make_async_copy(
                table_hbm.at[idx_v.at[j]], rows_v.at[b], gsems[b]
            ).wait()
            pltpu.async_copy(
                rows_v.at[b],
                out_hbm.at[j // _CPP, pl.ds(i_base + (j % _CPP) * _CH, _CH)],
                ssems[b],
            )

        @pl.when(g < _GROUPS - 1)
        def _():
            for b in range(_NB):
                jn = (g + 1) * _NB + b
                pltpu.make_async_copy(
                    rows_v.at[b], out_hbm.at[0, pl.ds(i_base, _CH)], ssems[b]
                ).wait()
                pltpu.async_copy(
                    table_hbm.at[idx_v.at[jn]], rows_v.at[b], gsems[b]
                )

        return carry

    lax.fori_loop(0, _GROUPS, group, 0)

    for b in range(_REM):
        jn = _GROUPS * _NB + b
        pltpu.make_async_copy(
            rows_v.at[b], out_hbm.at[0, pl.ds(i_base, _CH)], ssems[b]
        ).wait()
        pltpu.async_copy(table_hbm.at[idx_v.at[jn]], rows_v.at[b], gsems[b])
    for b in range(_REM):
        jn = _GROUPS * _NB + b
        pltpu.make_async_copy(
            table_hbm.at[idx_v.at[jn]], rows_v.at[b], gsems[b]
        ).wait()
        pltpu.async_copy(
            rows_v.at[b],
            out_hbm.at[jn // _CPP, pl.ds(i_base + (jn % _CPP) * _CH, _CH)],
            ssems[b],
        )

    for b in range(_NB):
        pltpu.make_async_copy(
            rows_v.at[b], out_hbm.at[0, pl.ds(i_base, _CH)], ssems[b]
        ).wait()


def kernel(x, embedding_weights):
    idx = x.astype(jnp.int32).reshape(_NW, _RPW, _SEQ).transpose(0, 2, 1)
    idx = idx.reshape(_NW, _N_CHUNKS, _CH)
    out = _sc_gather(idx, embedding_weights)
    return out.transpose(1, 0, 2)
